# baseline (speedup 1.0000x reference)
import functools

import jax
import jax.numpy as jnp
from jax import lax
from jax.experimental import pallas as pl
from jax.experimental.pallas import tpu as pltpu
from jax.experimental.pallas import tpu_sc as plsc

B = 1024
N_CLASSES = 100000
PLANES = 512
K = 5

NC = 2
NS = 16
LANES = 16
NW = NC * NS
ROWS_PER_W = B // NW
GROUP = 8
N_GROUPS = ROWS_PER_W // GROUP
N_SETS = 3
CAP = N_SETS * K * LANES
CHUNK = 1920
ALIGNED_COLS = 99840
SC_COLS = 57600
N_CHUNKS = SC_COLS // CHUNK
TC_COLS = ALIGNED_COLS - SC_COLS
TAIL = N_CLASSES - ALIGNED_COLS
VREGS_PER_ROW = CHUNK // LANES
ITERS = VREGS_PER_ROW // N_SETS


def _sc_candidates(logits):
  mesh = plsc.VectorSubcoreMesh(core_axis_name="c", subcore_axis_name="s")
  rows = B
  rows_per_w = rows // NW

  @functools.partial(
      pl.kernel,
      mesh=mesh,
      out_type=jax.ShapeDtypeStruct((rows, CAP), jnp.float32),
      scratch_types=[
          pltpu.VMEM((2, GROUP, CHUNK), jnp.float32),
          pltpu.VMEM((GROUP, CAP), jnp.float32),
          pltpu.SemaphoreType.DMA,
          pltpu.SemaphoreType.DMA,
      ],
  )
  def cand_kernel(logits_hbm, out_hbm, buf, acc, sem0, sem1):
    wid = lax.axis_index("s") * NC + lax.axis_index("c")
    sems = [sem0, sem1]
    neg_inf = jnp.full((LANES,), -jnp.inf, dtype=jnp.float32)

    def start_dma(rbase, c, par):
      pltpu.async_copy(
          logits_hbm.at[pl.ds(rbase, GROUP), pl.ds(c * CHUNK, CHUNK)],
          buf.at[par], sems[par])

    def wait_dma(rbase, par):
      pltpu.make_async_copy(
          logits_hbm.at[pl.ds(rbase, GROUP), pl.ds(0, CHUNK)],
          buf.at[par], sems[par]).wait()

    def process(rbase, par):
      def row_body(rr, _):
        a = tuple(acc[rr, pl.ds(j * LANES, LANES)] for j in range(N_SETS * K))

        def vbody(v, a):
          accs = list(a)
          for s in range(N_SETS):
            x = buf[par, rr, pl.ds((v * N_SETS + s) * LANES, LANES)]
            for k in range(K):
              t = accs[s * K + k]
              hi = jnp.maximum(t, x)
              x = jnp.minimum(t, x)
              accs[s * K + k] = hi
          return tuple(accs)

        a = lax.fori_loop(0, ITERS, vbody, a)
        for j in range(N_SETS * K):
          acc[rr, pl.ds(j * LANES, LANES)] = a[j]
        return 0

      lax.fori_loop(0, GROUP, row_body, 0)

    for g in range(rows_per_w // GROUP):
      rel = wid * rows_per_w + g * GROUP
      rbase = rel

      def init_body(rr, _):
        for j in range(N_SETS * K):
          acc[rr, pl.ds(j * LANES, LANES)] = neg_inf
        return 0

      lax.fori_loop(0, GROUP, init_body, 0)

      start_dma(rbase, 0, 0)
      start_dma(rbase, 1, 1)

      def chunk_body(c2, _, rbase=rbase):
        wait_dma(rbase, 0)
        process(rbase, 0)
        start_dma(rbase, 2 * c2 + 2, 0)
        wait_dma(rbase, 1)
        process(rbase, 1)
        start_dma(rbase, 2 * c2 + 3, 1)
        return 0

      lax.fori_loop(0, N_CHUNKS // 2 - 1, chunk_body, 0)
      wait_dma(rbase, 0)
      process(rbase, 0)
      wait_dma(rbase, 1)
      process(rbase, 1)

      pltpu.sync_copy(acc, out_hbm.at[pl.ds(rel, GROUP)])

  return cand_kernel(logits)


TCK_ROWS = 256
TCK_STEPS = TC_COLS // CHUNK
TC_CAP = K * 128


def _tck_body(x_ref, out_ref, acc_ref):
  j = pl.program_id(1)

  @pl.when(j == 0)
  def _():
    acc_ref[...] = jnp.full(acc_ref.shape, -jnp.inf, dtype=jnp.float32)

  x = x_ref[...]
  acc = [acc_ref[k] for k in range(K)]
  for s in range(CHUNK // 128):
    v = x[:, s * 128:(s + 1) * 128]
    for k in range(K):
      hi = jnp.maximum(acc[k], v)
      v = jnp.minimum(acc[k], v)
      acc[k] = hi
  for k in range(K):
    acc_ref[k] = acc[k]

  @pl.when(j == TCK_STEPS - 1)
  def _():
    out_ref[...] = jnp.concatenate(acc, axis=1)


def _tc_topk(tc_share):
  return pl.pallas_call(
      _tck_body,
      grid=(B // TCK_ROWS, TCK_STEPS),
      in_specs=[
          pl.BlockSpec((TCK_ROWS, CHUNK), lambda i, j: (i, j)),
      ],
      out_specs=pl.BlockSpec((TCK_ROWS, TC_CAP), lambda i, j: (i, 0)),
      out_shape=jax.ShapeDtypeStruct((B, TC_CAP), jnp.float32),
      scratch_shapes=[pltpu.VMEM((K, TCK_ROWS, 128), jnp.float32)],
  )(tc_share)


ROWS_BLK = 128
TOPW = CAP + TC_CAP + TAIL


def _tc_finish_body(cand_ref, tcc_ref, tail_ref, feat_ref, w1f_ref, w1t_ref,
                    b1_ref, w2_ref, b2_ref, out_ref):
  x = jnp.concatenate([cand_ref[...], tcc_ref[...], tail_ref[...]], axis=1)
  col = lax.broadcasted_iota(jnp.int32, x.shape, 1)
  tops = []
  for _ in range(K):
    m = jnp.max(x, axis=1, keepdims=True)
    hit = x == m
    first = jnp.min(jnp.where(hit, col, TOPW), axis=1, keepdims=True)
    x = jnp.where(col == first, -jnp.inf, x)
    tops.append(m)

  h = jnp.dot(feat_ref[...], w1f_ref[...], preferred_element_type=jnp.float32)
  for k in range(K):
    h = h + tops[k] * w1t_ref[k:k + 1, :]
  h = h + b1_ref[...]
  h = jnp.maximum(h, 0.0)
  o = jnp.sum(h * w2_ref[...], axis=1, keepdims=True) + b2_ref[...]
  out_ref[...] = 1.0 / (1.0 + jnp.exp(-o))


def _tc_finish(cands, tccands, tail, features, W1, b1, W2, b2):
  w1f = W1[:, :PLANES].T
  w1t = jnp.pad(W1[:, PLANES:].T, ((0, 3), (0, 0)))
  b1r = b1.reshape(1, -1)
  b2r = b2.reshape(1, 1)
  grid = (B // ROWS_BLK,)
  return pl.pallas_call(
      _tc_finish_body,
      grid=grid,
      in_specs=[
          pl.BlockSpec((ROWS_BLK, CAP), lambda i: (i, 0)),
          pl.BlockSpec((ROWS_BLK, TC_CAP), lambda i: (i, 0)),
          pl.BlockSpec((ROWS_BLK, TAIL), lambda i: (i, 0)),
          pl.BlockSpec((ROWS_BLK, PLANES), lambda i: (i, 0)),
          pl.BlockSpec((PLANES, 100), lambda i: (0, 0)),
          pl.BlockSpec((8, 100), lambda i: (0, 0)),
          pl.BlockSpec((1, 100), lambda i: (0, 0)),
          pl.BlockSpec((1, 100), lambda i: (0, 0)),
          pl.BlockSpec((1, 1), lambda i: (0, 0)),
      ],
      out_specs=pl.BlockSpec((ROWS_BLK, 1), lambda i: (i, 0)),
      out_shape=jax.ShapeDtypeStruct((B, 1), jnp.float32),
  )(cands, tccands, tail, features, w1f, w1t, b1r, W2, b2r)


def kernel(logits, features, W1, b1, W2, b2):
  sc_share = lax.slice(logits, (0, 0), (B, SC_COLS))
  cands = _sc_candidates(sc_share)
  tc_share = lax.slice(logits, (0, SC_COLS), (B, ALIGNED_COLS))
  tccands = _tc_topk(tc_share)
  tail = lax.slice(logits, (0, ALIGNED_COLS), (B, N_CLASSES))
  return _tc_finish(cands, tccands, tail, features, W1, b1, W2, b2)


# speedup vs baseline: 1.4490x; 1.4490x over previous
import functools

import jax
import jax.numpy as jnp
from jax import lax
from jax.experimental import pallas as pl
from jax.experimental.pallas import tpu as pltpu
from jax.experimental.pallas import tpu_sc as plsc

B = 1024
N_CLASSES = 100000
PLANES = 512
K = 5

NC = 2
NS = 16
LANES = 16
NW = NC * NS
ROWS_PER_W = B // NW
GROUP = 8
N_GROUPS = ROWS_PER_W // GROUP
N_SETS = 3
CAP = N_SETS * K * LANES
CHUNK = 1920
ALIGNED_COLS = 99840
SC_COLS = 42240
N_CHUNKS = SC_COLS // CHUNK
TC_COLS = ALIGNED_COLS - SC_COLS
TAIL = N_CLASSES - ALIGNED_COLS
VREGS_PER_ROW = CHUNK // LANES
ITERS = VREGS_PER_ROW // N_SETS


def _sc_candidates(logits):
  mesh = plsc.VectorSubcoreMesh(core_axis_name="c", subcore_axis_name="s")
  rows = B
  rows_per_w = rows // NW

  @functools.partial(
      pl.kernel,
      mesh=mesh,
      out_type=jax.ShapeDtypeStruct((rows, CAP), jnp.float32),
      scratch_types=[
          pltpu.VMEM((2, GROUP, CHUNK), jnp.float32),
          pltpu.VMEM((GROUP, CAP), jnp.float32),
          pltpu.SemaphoreType.DMA,
          pltpu.SemaphoreType.DMA,
      ],
  )
  def cand_kernel(logits_hbm, out_hbm, buf, acc, sem0, sem1):
    wid = lax.axis_index("s") * NC + lax.axis_index("c")
    sems = [sem0, sem1]
    neg_inf = jnp.full((LANES,), -jnp.inf, dtype=jnp.float32)

    def start_dma(rbase, c, par):
      pltpu.async_copy(
          logits_hbm.at[pl.ds(rbase, GROUP), pl.ds(c * CHUNK, CHUNK)],
          buf.at[par], sems[par])

    def wait_dma(rbase, par):
      pltpu.make_async_copy(
          logits_hbm.at[pl.ds(rbase, GROUP), pl.ds(0, CHUNK)],
          buf.at[par], sems[par]).wait()

    def process(rbase, par):
      def row_body(rr, _):
        a = tuple(acc[rr, pl.ds(j * LANES, LANES)] for j in range(N_SETS * K))

        def vbody(v, a):
          accs = list(a)
          for s in range(N_SETS):
            x = buf[par, rr, pl.ds((v * N_SETS + s) * LANES, LANES)]
            for k in range(K):
              t = accs[s * K + k]
              hi = jnp.maximum(t, x)
              x = jnp.minimum(t, x)
              accs[s * K + k] = hi
          return tuple(accs)

        a = lax.fori_loop(0, ITERS, vbody, a)
        for j in range(N_SETS * K):
          acc[rr, pl.ds(j * LANES, LANES)] = a[j]
        return 0

      lax.fori_loop(0, GROUP, row_body, 0)

    for g in range(rows_per_w // GROUP):
      rel = wid * rows_per_w + g * GROUP
      rbase = rel

      def init_body(rr, _):
        for j in range(N_SETS * K):
          acc[rr, pl.ds(j * LANES, LANES)] = neg_inf
        return 0

      lax.fori_loop(0, GROUP, init_body, 0)

      start_dma(rbase, 0, 0)
      start_dma(rbase, 1, 1)

      def chunk_body(c2, _, rbase=rbase):
        wait_dma(rbase, 0)
        process(rbase, 0)
        start_dma(rbase, 2 * c2 + 2, 0)
        wait_dma(rbase, 1)
        process(rbase, 1)
        start_dma(rbase, 2 * c2 + 3, 1)
        return 0

      lax.fori_loop(0, N_CHUNKS // 2 - 1, chunk_body, 0)
      wait_dma(rbase, 0)
      process(rbase, 0)
      wait_dma(rbase, 1)
      process(rbase, 1)

      pltpu.sync_copy(acc, out_hbm.at[pl.ds(rel, GROUP)])

  return cand_kernel(logits)


TCK_ROWS = 256
TCK_STEPS = TC_COLS // CHUNK
TC_CAP = K * 128


def _tck_body(x_ref, out_ref, acc_ref):
  j = pl.program_id(1)

  @pl.when(j == 0)
  def _():
    acc_ref[...] = jnp.full(acc_ref.shape, -jnp.inf, dtype=jnp.float32)

  x = x_ref[...]
  acc = [acc_ref[k] for k in range(K)]
  for s in range(CHUNK // 128):
    v = x[:, s * 128:(s + 1) * 128]
    for k in range(K):
      hi = jnp.maximum(acc[k], v)
      v = jnp.minimum(acc[k], v)
      acc[k] = hi
  for k in range(K):
    acc_ref[k] = acc[k]

  @pl.when(j == TCK_STEPS - 1)
  def _():
    out_ref[...] = jnp.concatenate(acc, axis=1)


def _tc_topk(logits):
  return pl.pallas_call(
      _tck_body,
      grid=(B // TCK_ROWS, TCK_STEPS),
      in_specs=[
          pl.BlockSpec((TCK_ROWS, CHUNK),
                       lambda i, j: (i, j + SC_COLS // CHUNK)),
      ],
      out_specs=pl.BlockSpec((TCK_ROWS, TC_CAP), lambda i, j: (i, 0)),
      out_shape=jax.ShapeDtypeStruct((B, TC_CAP), jnp.float32),
      scratch_shapes=[pltpu.VMEM((K, TCK_ROWS, 128), jnp.float32)],
  )(logits)


ROWS_BLK = 128
TOPW = CAP + TC_CAP + TAIL


def _tc_finish_body(cand_ref, tcc_ref, tail_ref, feat_ref, w1f_ref, w1t_ref,
                    b1_ref, w2_ref, b2_ref, out_ref):
  x = jnp.concatenate([cand_ref[...], tcc_ref[...], tail_ref[...]], axis=1)
  col = lax.broadcasted_iota(jnp.int32, x.shape, 1)
  tops = []
  for _ in range(K):
    m = jnp.max(x, axis=1, keepdims=True)
    hit = x == m
    first = jnp.min(jnp.where(hit, col, TOPW), axis=1, keepdims=True)
    x = jnp.where(col == first, -jnp.inf, x)
    tops.append(m)

  h = jnp.dot(feat_ref[...], w1f_ref[...], preferred_element_type=jnp.float32)
  for k in range(K):
    h = h + tops[k] * w1t_ref[k:k + 1, :]
  h = h + b1_ref[...]
  h = jnp.maximum(h, 0.0)
  o = jnp.sum(h * w2_ref[...], axis=1, keepdims=True) + b2_ref[...]
  out_ref[...] = 1.0 / (1.0 + jnp.exp(-o))


def _tc_finish(cands, tccands, tail, features, W1, b1, W2, b2):
  w1f = W1[:, :PLANES].T
  w1t = jnp.pad(W1[:, PLANES:].T, ((0, 3), (0, 0)))
  b1r = b1.reshape(1, -1)
  b2r = b2.reshape(1, 1)
  grid = (B // ROWS_BLK,)
  return pl.pallas_call(
      _tc_finish_body,
      grid=grid,
      in_specs=[
          pl.BlockSpec((ROWS_BLK, CAP), lambda i: (i, 0)),
          pl.BlockSpec((ROWS_BLK, TC_CAP), lambda i: (i, 0)),
          pl.BlockSpec((ROWS_BLK, TAIL), lambda i: (i, 0)),
          pl.BlockSpec((ROWS_BLK, PLANES), lambda i: (i, 0)),
          pl.BlockSpec((PLANES, 100), lambda i: (0, 0)),
          pl.BlockSpec((8, 100), lambda i: (0, 0)),
          pl.BlockSpec((1, 100), lambda i: (0, 0)),
          pl.BlockSpec((1, 100), lambda i: (0, 0)),
          pl.BlockSpec((1, 1), lambda i: (0, 0)),
      ],
      out_specs=pl.BlockSpec((ROWS_BLK, 1), lambda i: (i, 0)),
      out_shape=jax.ShapeDtypeStruct((B, 1), jnp.float32),
  )(cands, tccands, tail, features, w1f, w1t, b1r, W2, b2r)


def kernel(logits, features, W1, b1, W2, b2):
  cands = _sc_candidates(logits)
  tccands = _tc_topk(logits)
  tail = lax.slice(logits, (0, ALIGNED_COLS), (B, N_CLASSES))
  return _tc_finish(cands, tccands, tail, features, W1, b1, W2, b2)
